# baseline (speedup 1.0000x reference)
import jax
import jax.numpy as jnp
from jax.experimental import pallas as pl
from jax.experimental.pallas import tpu as pltpu

B = 16
C_IN = 3
H = 512
W = 512
HW = H * W
NUM_EXPERTS = 3
HIDDEN = 768
OUT_L = 200
OUT_B = 400

RC = 16
NCH = H // RC
NBUF = 10


def _moe_kernel(pix_hbm, Wc_ref, bc_ref, W1_ref, b1_ref, W2l_hbm, W2b_hbm,
                logits_ref, boxes_ref, bufs, w2l_buf, w2b_buf, psems, wsems):

    def pix_copy(c):
        return pltpu.make_async_copy(
            pix_hbm.at[:, :, pl.ds(c * RC, RC), :], bufs.at[c % NBUF],
            psems.at[c % NBUF])

    def w2_copies():
        return (pltpu.make_async_copy(W2l_hbm, w2l_buf, wsems.at[0]),
                pltpu.make_async_copy(W2b_hbm, w2b_buf, wsems.at[1]))

    for j in range(NBUF):
        pix_copy(j).start()
    for cp in w2_copies():
        cp.start()

    acc = jnp.zeros((B, C_IN), jnp.float32)
    for c in range(NCH):
        pix_copy(c).wait()
        acc = acc + jnp.sum(bufs[c % NBUF], axis=(2, 3))
        if c + NBUF < NCH:
            pix_copy(c + NBUF).start()

    pooled = acc * (1.0 / HW)
    dl = jnp.dot(pooled, Wc_ref[...],
                 preferred_element_type=jnp.float32) + bc_ref[...]
    l0 = dl[:, 0:1]
    l1 = dl[:, 1:2]
    l2 = dl[:, 2:3]
    w0 = ((l0 >= l1) & (l0 >= l2)).astype(jnp.float32)
    w1 = ((l1 > l0) & (l1 >= l2)).astype(jnp.float32)
    w2 = ((l2 > l0) & (l2 > l1)).astype(jnp.float32)
    masks = (w0, w1, w2)

    for cp in w2_copies():
        cp.wait()
    acc_l = jnp.zeros((B, OUT_L), dtype=jnp.float32)
    acc_b = jnp.zeros((B, OUT_B), dtype=jnp.float32)
    for e in range(NUM_EXPERTS):
        h = jax.nn.relu(
            jnp.dot(pooled, W1_ref[e],
                    preferred_element_type=jnp.float32) + b1_ref[e:e + 1])
        hm = h * masks[e]
        acc_l += jax.lax.dot_general(
            hm, w2l_buf[e], (((1,), (1,)), ((), ())),
            preferred_element_type=jnp.float32)
        acc_b += jax.lax.dot_general(
            hm, w2b_buf[e], (((1,), (1,)), ((), ())),
            preferred_element_type=jnp.float32)
    logits_ref[...] = acc_l
    boxes_ref[...] = jax.nn.sigmoid(acc_b)


@jax.jit
def kernel(pixel_values, Wc, bc, W1, b1, W2l, W2b):
    bc2 = bc.reshape(1, NUM_EXPERTS)
    logits, boxes = pl.pallas_call(
        _moe_kernel,
        in_specs=[
            pl.BlockSpec(memory_space=pl.ANY),
            pl.BlockSpec((C_IN, NUM_EXPERTS), lambda: (0, 0)),
            pl.BlockSpec((1, NUM_EXPERTS), lambda: (0, 0)),
            pl.BlockSpec((NUM_EXPERTS, C_IN, HIDDEN), lambda: (0, 0, 0)),
            pl.BlockSpec((NUM_EXPERTS, HIDDEN), lambda: (0, 0)),
            pl.BlockSpec(memory_space=pl.ANY),
            pl.BlockSpec(memory_space=pl.ANY),
        ],
        out_specs=[
            pl.BlockSpec((B, OUT_L), lambda: (0, 0)),
            pl.BlockSpec((B, OUT_B), lambda: (0, 0)),
        ],
        out_shape=[
            jax.ShapeDtypeStruct((B, OUT_L), jnp.float32),
            jax.ShapeDtypeStruct((B, OUT_B), jnp.float32),
        ],
        scratch_shapes=[
            pltpu.VMEM((NBUF, B, C_IN, RC, W), jnp.float32),
            pltpu.VMEM((NUM_EXPERTS, OUT_L, HIDDEN), jnp.float32),
            pltpu.VMEM((NUM_EXPERTS, OUT_B, HIDDEN), jnp.float32),
            pltpu.SemaphoreType.DMA((NBUF,)),
            pltpu.SemaphoreType.DMA((2,)),
        ],
    )(pixel_values, Wc, bc2, W1, b1,
      jnp.swapaxes(W2l, 1, 2), jnp.swapaxes(W2b, 1, 2))
    return logits.reshape(B, 100, 2), boxes.reshape(B, 100, 4)


# speedup vs baseline: 1.0607x; 1.0607x over previous
import jax
import jax.numpy as jnp
from jax.experimental import pallas as pl
from jax.experimental.pallas import tpu as pltpu

B = 16
C_IN = 3
H = 512
W = 512
HW = H * W
NUM_EXPERTS = 3
HIDDEN = 768
OUT_L = 200
OUT_B = 400

HCHUNK = 64
GRID = H // HCHUNK


def _moe_kernel(pix_ref, Wc_ref, bc_ref, W1_ref, b1_ref, W2l_hbm, W2b_hbm,
                logits_ref, boxes_ref, acc_ref, w2l_buf, w2b_buf, sems):

    def w2_copies():
        return (pltpu.make_async_copy(W2l_hbm, w2l_buf, sems.at[0]),
                pltpu.make_async_copy(W2b_hbm, w2b_buf, sems.at[1]))

    i = pl.program_id(0)

    @pl.when(i == 0)
    def _init():
        acc_ref[...] = jnp.zeros_like(acc_ref)
        for cp in w2_copies():
            cp.start()

    acc_ref[...] += jnp.sum(pix_ref[...], axis=(2, 3))

    @pl.when(i == GRID - 1)
    def _finish():
        for cp in w2_copies():
            cp.wait()
        pooled = acc_ref[...] * (1.0 / HW)
        dl = jnp.dot(pooled, Wc_ref[...],
                     preferred_element_type=jnp.float32) + bc_ref[...]
        l0 = dl[:, 0:1]
        l1 = dl[:, 1:2]
        l2 = dl[:, 2:3]
        w0 = ((l0 >= l1) & (l0 >= l2)).astype(jnp.float32)
        w1 = ((l1 > l0) & (l1 >= l2)).astype(jnp.float32)
        w2 = ((l2 > l0) & (l2 > l1)).astype(jnp.float32)
        masks = (w0, w1, w2)

        acc_l = jnp.zeros((B, OUT_L), dtype=jnp.float32)
        acc_b = jnp.zeros((B, OUT_B), dtype=jnp.float32)
        for e in range(NUM_EXPERTS):
            h = jax.nn.relu(
                jnp.dot(pooled, W1_ref[e],
                        preferred_element_type=jnp.float32) + b1_ref[e:e + 1])
            hm = h * masks[e]
            acc_l += jax.lax.dot_general(
                hm, w2l_buf[e], (((1,), (1,)), ((), ())),
                preferred_element_type=jnp.float32)
            acc_b += jax.lax.dot_general(
                hm, w2b_buf[e], (((1,), (1,)), ((), ())),
                preferred_element_type=jnp.float32)
        logits_ref[...] = acc_l
        boxes_ref[...] = jax.nn.sigmoid(acc_b)


@jax.jit
def kernel(pixel_values, Wc, bc, W1, b1, W2l, W2b):
    bc2 = bc.reshape(1, NUM_EXPERTS)
    logits, boxes = pl.pallas_call(
        _moe_kernel,
        grid=(GRID,),
        in_specs=[
            pl.BlockSpec((B, C_IN, HCHUNK, W), lambda i: (0, 0, i, 0)),
            pl.BlockSpec((C_IN, NUM_EXPERTS), lambda i: (0, 0)),
            pl.BlockSpec((1, NUM_EXPERTS), lambda i: (0, 0)),
            pl.BlockSpec((NUM_EXPERTS, C_IN, HIDDEN), lambda i: (0, 0, 0)),
            pl.BlockSpec((NUM_EXPERTS, HIDDEN), lambda i: (0, 0)),
            pl.BlockSpec(memory_space=pl.ANY),
            pl.BlockSpec(memory_space=pl.ANY),
        ],
        out_specs=[
            pl.BlockSpec((B, OUT_L), lambda i: (0, 0)),
            pl.BlockSpec((B, OUT_B), lambda i: (0, 0)),
        ],
        out_shape=[
            jax.ShapeDtypeStruct((B, OUT_L), jnp.float32),
            jax.ShapeDtypeStruct((B, OUT_B), jnp.float32),
        ],
        scratch_shapes=[
            pltpu.VMEM((B, C_IN), jnp.float32),
            pltpu.VMEM((NUM_EXPERTS, OUT_L, HIDDEN), jnp.float32),
            pltpu.VMEM((NUM_EXPERTS, OUT_B, HIDDEN), jnp.float32),
            pltpu.SemaphoreType.DMA((2,)),
        ],
    )(pixel_values, Wc, bc2, W1, b1,
      jnp.swapaxes(W2l, 1, 2), jnp.swapaxes(W2b, 1, 2))
    return logits.reshape(B, 100, 2), boxes.reshape(B, 100, 4)
